# Initial kernel scaffold; baseline (speedup 1.0000x reference)
import jax
import jax.numpy as jnp
from jax.experimental import pallas as pl


def kernel(field_0, field_1, field_2, table):
    raise NotImplementedError("write your pallas kernel here")


# speedup vs baseline: 1.6751x; 1.6751x over previous
import functools

import jax
import jax.numpy as jnp
from jax import lax
from jax.experimental import pallas as pl
from jax.experimental.pallas import tpu as pltpu
from jax.experimental.pallas import tpu_sc as plsc

EMBED_DIM = 32
ROWS_PER_STREAM = 128
K = 8
CHUNK = K * ROWS_PER_STREAM


def _make_gather(total_rows: int):
    info = plsc.get_sparse_core_info()
    nc, ns = info.num_cores, info.num_subcores
    nw = nc * ns
    assert total_rows % (nw * CHUNK) == 0
    rows_per_w = total_rows // nw
    chunks_per_w = rows_per_w // CHUNK
    idx_rows = total_rows // ROWS_PER_STREAM

    mesh = plsc.VectorSubcoreMesh(core_axis_name="c", subcore_axis_name="s")

    @functools.partial(
        pl.kernel,
        mesh=mesh,
        out_type=jax.ShapeDtypeStruct((total_rows, EMBED_DIM), jnp.float32),
        scratch_types=[
            pltpu.VMEM((K, ROWS_PER_STREAM), jnp.int32),
            pltpu.VMEM((CHUNK, EMBED_DIM), jnp.float32),
            pltpu.SemaphoreType.DMA,
        ],
        compiler_params=pltpu.CompilerParams(use_tc_tiling_on_sc=False),
    )
    def gather_kernel(table_hbm, idx_hbm, out_hbm, idx_v, rows_v, sem):
        wid = lax.axis_index("s") * nc + lax.axis_index("c")
        w_idx_row0 = wid * (rows_per_w // ROWS_PER_STREAM)

        def body(g, carry):
            irow0 = w_idx_row0 + g * K
            pltpu.sync_copy(idx_hbm.at[pl.ds(irow0, K)], idx_v)
            cps = [
                pltpu.async_copy(
                    table_hbm.at[idx_v.at[j]],
                    rows_v.at[pl.ds(j * ROWS_PER_STREAM, ROWS_PER_STREAM)],
                    sem,
                )
                for j in range(K)
            ]
            for c in cps:
                c.wait()
            pltpu.sync_copy(
                rows_v, out_hbm.at[pl.ds(irow0 * ROWS_PER_STREAM, CHUNK)]
            )
            return carry

        lax.fori_loop(0, chunks_per_w, body, 0)

    def run(table, idx_flat):
        idx2d = idx_flat.reshape(idx_rows, ROWS_PER_STREAM)
        return gather_kernel(table, idx2d)

    return run


@jax.jit
def kernel(field_0, field_1, field_2, table):
    b, l = field_0.shape
    total = 3 * b * l
    idx_flat = jnp.stack(
        [
            field_0.reshape(-1).astype(jnp.int32),
            field_1.reshape(-1).astype(jnp.int32),
            field_2.reshape(-1).astype(jnp.int32),
        ],
        axis=1,
    ).reshape(-1)
    out_flat = _make_gather(total)(table, idx_flat)
    return out_flat.reshape(b, l, 3 * EMBED_DIM)


# speedup vs baseline: 1.7158x; 1.0243x over previous
import functools

import jax
import jax.numpy as jnp
from jax import lax
from jax.experimental import pallas as pl
from jax.experimental.pallas import tpu as pltpu
from jax.experimental.pallas import tpu_sc as plsc

EMBED_DIM = 32
ROWS_PER_STREAM = 128
K = 8
CHUNK = K * ROWS_PER_STREAM


def _make_gather(total_rows: int):
    info = plsc.get_sparse_core_info()
    nc, ns = info.num_cores, info.num_subcores
    nw = nc * ns
    assert total_rows % (nw * CHUNK) == 0
    rows_per_w = total_rows // nw
    chunks_per_w = rows_per_w // CHUNK
    n_paired = chunks_per_w // 2
    n_peeled = chunks_per_w - 2 * n_paired
    idx_rows = total_rows // ROWS_PER_STREAM

    mesh = plsc.VectorSubcoreMesh(core_axis_name="c", subcore_axis_name="s")

    @functools.partial(
        pl.kernel,
        mesh=mesh,
        out_type=jax.ShapeDtypeStruct((total_rows, EMBED_DIM), jnp.float32),
        scratch_types=[
            pltpu.VMEM((2, K, ROWS_PER_STREAM), jnp.int32),
            pltpu.VMEM((2, CHUNK, EMBED_DIM), jnp.float32),
            pltpu.SemaphoreType.DMA,
            pltpu.SemaphoreType.DMA,
            pltpu.SemaphoreType.DMA,
        ],
        compiler_params=pltpu.CompilerParams(use_tc_tiling_on_sc=False),
    )
    def gather_kernel(table_hbm, idx_hbm, out_hbm, idx_v, rows_v, sem_g,
                      sem_o0, sem_o1):
        wid = lax.axis_index("s") * nc + lax.axis_index("c")
        w_idx_row0 = wid * (rows_per_w // ROWS_PER_STREAM)
        sem_o = (sem_o0, sem_o1)

        def do_chunk(c, b):
            irow0 = w_idx_row0 + c * K
            pltpu.sync_copy(idx_hbm.at[pl.ds(irow0, K)], idx_v.at[b])
            cps = [
                pltpu.async_copy(
                    table_hbm.at[idx_v.at[b, j]],
                    rows_v.at[b, pl.ds(j * ROWS_PER_STREAM, ROWS_PER_STREAM)],
                    sem_g,
                )
                for j in range(K)
            ]
            for c_ in cps:
                c_.wait()
            pltpu.async_copy(
                rows_v.at[b],
                out_hbm.at[pl.ds(irow0 * ROWS_PER_STREAM, CHUNK)],
                sem_o[b],
            )

        def body(h, carry):
            for b in range(2):
                c = 2 * h + b

                @pl.when(h >= 1)
                def _():
                    pltpu.make_async_copy(
                        rows_v.at[b],
                        out_hbm.at[pl.ds(0, CHUNK)],
                        sem_o[b],
                    ).wait()

                do_chunk(c, b)
            return carry

        lax.fori_loop(0, n_paired, body, 0)

        if n_peeled:
            pltpu.make_async_copy(
                rows_v.at[0], out_hbm.at[pl.ds(0, CHUNK)], sem_o[0]
            ).wait()
            do_chunk(2 * n_paired, 0)

        for b in range(2):
            pltpu.make_async_copy(
                rows_v.at[b], out_hbm.at[pl.ds(0, CHUNK)], sem_o[b]
            ).wait()

    def run(table, idx_flat):
        idx2d = idx_flat.reshape(idx_rows, ROWS_PER_STREAM)
        return gather_kernel(table, idx2d)

    return run


@jax.jit
def kernel(field_0, field_1, field_2, table):
    b, l = field_0.shape
    total = 3 * b * l
    idx_flat = jnp.stack(
        [
            field_0.reshape(-1).astype(jnp.int32),
            field_1.reshape(-1).astype(jnp.int32),
            field_2.reshape(-1).astype(jnp.int32),
        ],
        axis=1,
    ).reshape(-1)
    out_flat = _make_gather(total)(table, idx_flat)
    return out_flat.reshape(b, l, 3 * EMBED_DIM)


# speedup vs baseline: 1.7402x; 1.0142x over previous
import functools

import jax
import jax.numpy as jnp
from jax import lax
from jax.experimental import pallas as pl
from jax.experimental.pallas import tpu as pltpu
from jax.experimental.pallas import tpu_sc as plsc

EMBED_DIM = 32
ROWS_PER_STREAM = 128
K = 8
CHUNK = K * ROWS_PER_STREAM


def _make_gather(total_rows: int):
    info = plsc.get_sparse_core_info()
    nc, ns = info.num_cores, info.num_subcores
    nw = nc * ns
    assert total_rows % (nw * CHUNK) == 0
    rows_per_w = total_rows // nw
    n = rows_per_w // CHUNK
    assert n % 2 == 1 and n >= 3
    n_mid_pairs = (n - 3) // 2
    idx_rows = total_rows // ROWS_PER_STREAM

    mesh = plsc.VectorSubcoreMesh(core_axis_name="c", subcore_axis_name="s")

    @functools.partial(
        pl.kernel,
        mesh=mesh,
        out_type=jax.ShapeDtypeStruct((total_rows, EMBED_DIM), jnp.float32),
        scratch_types=[
            pltpu.VMEM((K, ROWS_PER_STREAM), jnp.int32),
            pltpu.VMEM((K, ROWS_PER_STREAM), jnp.int32),
            pltpu.VMEM((CHUNK, EMBED_DIM), jnp.float32),
            pltpu.VMEM((CHUNK, EMBED_DIM), jnp.float32),
            pltpu.SemaphoreType.DMA,
            pltpu.SemaphoreType.DMA,
            pltpu.SemaphoreType.DMA,
            pltpu.SemaphoreType.DMA,
            pltpu.SemaphoreType.DMA,
            pltpu.SemaphoreType.DMA,
        ],
        compiler_params=pltpu.CompilerParams(use_tc_tiling_on_sc=False),
    )
    def gather_kernel(table_hbm, idx_hbm, out_hbm, idx_v0, idx_v1, rows_v0,
                      rows_v1, sem_i0, sem_i1, sem_g0, sem_g1, sem_o0,
                      sem_o1):
        wid = lax.axis_index("s") * nc + lax.axis_index("c")
        w_idx_row0 = wid * (rows_per_w // ROWS_PER_STREAM)
        idx_v = (idx_v0, idx_v1)
        rows_v = (rows_v0, rows_v1)
        sem_i = (sem_i0, sem_i1)
        sem_g = (sem_g0, sem_g1)
        sem_o = (sem_o0, sem_o1)

        def issue_idx(c, b):
            pltpu.async_copy(
                idx_hbm.at[pl.ds(w_idx_row0 + c * K, K)], idx_v[b], sem_i[b]
            )

        def wait_idx(b):
            pltpu.make_async_copy(
                idx_hbm.at[pl.ds(0, K)], idx_v[b], sem_i[b]
            ).wait()

        def fire_gathers(b):
            for j in range(K):
                pltpu.async_copy(
                    table_hbm.at[idx_v[b].at[j]],
                    rows_v[b].at[pl.ds(j * ROWS_PER_STREAM, ROWS_PER_STREAM)],
                    sem_g[b],
                )

        def drain_gathers(b):
            pltpu.make_async_copy(
                table_hbm.at[pl.ds(0, CHUNK)], rows_v[b], sem_g[b]
            ).wait()

        def fire_out(c, b):
            pltpu.async_copy(
                rows_v[b],
                out_hbm.at[pl.ds((w_idx_row0 + c * K) * ROWS_PER_STREAM, CHUNK)],
                sem_o[b],
            )

        def wait_out(b):
            pltpu.make_async_copy(
                rows_v[b], out_hbm.at[pl.ds(0, CHUNK)], sem_o[b]
            ).wait()

        pltpu.sync_copy(idx_hbm.at[pl.ds(w_idx_row0, K)], idx_v[0])
        fire_gathers(0)
        issue_idx(1, 1)

        wait_idx(1)
        fire_gathers(1)
        drain_gathers(0)
        issue_idx(2, 0)
        fire_out(0, 0)

        def body(h, carry):
            for b, c_off in ((1, 1), (0, 2)):
                c = 2 * h + c_off
                wait_out(1 - b)
                wait_idx(1 - b)
                fire_gathers(1 - b)
                drain_gathers(b)
                issue_idx(c + 2, b)
                fire_out(c, b)
            return carry

        lax.fori_loop(0, n_mid_pairs, body, 0)

        wait_out(0)
        wait_idx(0)
        fire_gathers(0)
        drain_gathers(1)
        fire_out(n - 2, 1)

        wait_out(1)
        drain_gathers(0)
        fire_out(n - 1, 0)
        wait_out(0)

    def run(table, idx_flat):
        idx2d = idx_flat.reshape(idx_rows, ROWS_PER_STREAM)
        return gather_kernel(table, idx2d)

    return run


@jax.jit
def kernel(field_0, field_1, field_2, table):
    b, l = field_0.shape
    total = 3 * b * l
    idx_flat = jnp.stack(
        [
            field_0.reshape(-1).astype(jnp.int32),
            field_1.reshape(-1).astype(jnp.int32),
            field_2.reshape(-1).astype(jnp.int32),
        ],
        axis=1,
    ).reshape(-1)
    out_flat = _make_gather(total)(table, idx_flat)
    return out_flat.reshape(b, l, 3 * EMBED_DIM)
